# baseline (speedup 1.0000x reference)
import functools

import jax
import jax.numpy as jnp
from jax import lax
from jax.experimental import pallas as pl
from jax.experimental.pallas import tpu as pltpu
from jax.experimental.pallas import tpu_sc as plsc

B = 16384
D = 128
WALK = 5
NC = 2
NS = 16
NW = NC * NS
NB = B // NW
C = 64
NCHUNK = NB // C


def _sc_body(s_hbm, w_hbm, x_hbm, t_hbm, st_hbm, sidx, widx2, wix0, wix1,
             xs0, xs1, xw0, xw1, lt_v, t_v, st_v, sem0, sem1):
    wid = lax.axis_index("s") * NC + lax.axis_index("c")
    base = wid * NB
    pltpu.sync_copy(s_hbm.at[pl.ds(base, NB)], sidx)
    lane16 = lax.iota(jnp.int32, 16)

    xs_bufs = (xs0, xs1)
    xw_bufs = (xw0, xw1)
    wix_bufs = (wix0, wix1)
    sems = (sem0, sem1)

    def fire(c):
        xs_v, xw_v, sem = xs_bufs[c % 2], xw_bufs[c % 2], sems[c % 2]
        wix = wix_bufs[c % 2]
        pltpu.sync_copy(w_hbm.at[pl.ds(base + c * C, C), :], widx2)
        for g in range(C // 16):
            rows = g * 16 + lane16
            for j in range(WALK):
                cols = jnp.full((16,), j, jnp.int32)
                wix[j, pl.ds(g * 16, 16)] = plsc.load_gather(
                    widx2, [rows, cols])
        cps = [pltpu.async_copy(x_hbm.at[sidx.at[pl.ds(c * C, C)]], xs_v, sem)]
        for j in range(WALK):
            cps.append(pltpu.async_copy(
                x_hbm.at[wix.at[j, pl.ds(0, C)]],
                xw_v.at[pl.ds(j * C, C)], sem))
        return cps

    lane_iota = lax.iota(jnp.int32, 16)

    def compute(c):
        xs_v, xw_v = xs_bufs[c % 2], xw_bufs[c % 2]

        def g_body(g, carry2):
            def l_body(i, accs):
                b = g * 16 + i
                xs = [xs_v[b, pl.ds(k * 16, 16)] for k in range(8)]
                new = []
                for j in range(WALK):
                    r = j * C + b
                    acc = xs[0] * xw_v[r, pl.ds(0, 16)]
                    for k in range(1, 8):
                        acc = acc + xs[k] * xw_v[r, pl.ds(k * 16, 16)]
                    lj = jnp.sum(acc)
                    new.append(jnp.where(lane_iota == i, lj, accs[j]))
                return tuple(new)

            accs = lax.fori_loop(
                0, 16, l_body,
                tuple(jnp.zeros((16,), jnp.float32) for _ in range(WALK)))
            for j in range(WALK):
                lt_v[j, pl.ds(c * C + g * 16, 16)] = accs[j]
            return carry2

        lax.fori_loop(0, C // 16, g_body, 0)

    inflight = fire(0)
    for c in range(NCHUNK):
        for cp in inflight:
            cp.wait()
        if c + 1 < NCHUNK:
            inflight = fire(c + 1)
        compute(c)

    stats = []
    for j in range(WALK):
        m = lt_v[j, pl.ds(0, 16)]
        for i in range(1, NB // 16):
            m = jnp.maximum(m, lt_v[j, pl.ds(i * 16, 16)])
        mj = jnp.max(m)
        se = jnp.zeros((16,), jnp.float32)
        for i in range(NB // 16):
            se = se + jnp.exp(lt_v[j, pl.ds(i * 16, 16)] - mj)
        stats.append((mj, jnp.sum(se)))

    def t_body(i, carry3):
        tv = lt_v[0, pl.ds(i * 16, 16)]
        for j in range(1, WALK):
            tv = tv + lt_v[j, pl.ds(i * 16, 16)]
        t_v[pl.ds(i * 16, 16)] = tv
        return carry3

    lax.fori_loop(0, NB // 16, t_body, 0)
    pltpu.sync_copy(t_v, t_hbm.at[0, pl.ds(base, NB)])

    st = jnp.zeros((16,), jnp.float32)
    for j in range(WALK):
        st = jnp.where(lane_iota == j, stats[j][0], st)
        st = jnp.where(lane_iota == (j + 8), stats[j][1], st)
    for q in range(8):
        st_v[0, pl.ds(q * 16, 16)] = st if q == 0 else jnp.zeros(
            (16,), jnp.float32)
    pltpu.sync_copy(st_v, st_hbm.at[wid])


_sc_partial = functools.partial(
    pl.kernel,
    mesh=plsc.VectorSubcoreMesh(core_axis_name="c", subcore_axis_name="s"),
    compiler_params=pltpu.CompilerParams(needs_layout_passes=False),
    out_type=(
        jax.ShapeDtypeStruct((1, B), jnp.float32),
        jax.ShapeDtypeStruct((NW, 1, 128), jnp.float32),
    ),
    scratch_types=[
        pltpu.VMEM((NB,), jnp.int32),
        pltpu.VMEM((C, WALK), jnp.int32),
        pltpu.VMEM((WALK, C), jnp.int32),
        pltpu.VMEM((WALK, C), jnp.int32),
        pltpu.VMEM((C, D), jnp.float32),
        pltpu.VMEM((C, D), jnp.float32),
        pltpu.VMEM((C * WALK, D), jnp.float32),
        pltpu.VMEM((C * WALK, D), jnp.float32),
        pltpu.VMEM((WALK, NB), jnp.float32),
        pltpu.VMEM((NB,), jnp.float32),
        pltpu.VMEM((1, 128), jnp.float32),
        pltpu.SemaphoreType.DMA,
        pltpu.SemaphoreType.DMA,
    ],
)(_sc_body)


def _tc_body(t_ref, st_ref, out_ref):
    st = st_ref[...][:, 0, :]
    m = st[:, 0:WALK]
    se = st[:, 8:8 + WALK]
    gm = jnp.max(m, axis=0, keepdims=True)
    s_all = jnp.sum(se * jnp.exp(m - gm), axis=0, keepdims=True)
    k_const = jnp.sum(gm + jnp.log(s_all))
    out_ref[...] = k_const - t_ref[0]


def kernel(s, w, neg, X):
    del neg
    t, st = _sc_partial(s, w, X)
    return pl.pallas_call(
        _tc_body,
        out_shape=jax.ShapeDtypeStruct((B,), jnp.float32),
    )(t, st)


# speedup vs baseline: 1.1072x; 1.1072x over previous
import functools

import jax
import jax.numpy as jnp
from jax import lax
from jax.experimental import pallas as pl
from jax.experimental.pallas import tpu as pltpu
from jax.experimental.pallas import tpu_sc as plsc

B = 16384
D = 128
WALK = 5
NC = 2
NS = 16
NW = NC * NS
NB = B // NW
C = 64
NCHUNK = NB // C


def _sc_body(s_hbm, w_hbm, x_hbm, t_hbm, st_hbm, sidx, wst0, wst1, wix0, wix1,
             xs0, xs1, xw0, xw1, lt_v, t_v, st_v, sem0, sem1, wsem0, wsem1):
    wid = lax.axis_index("s") * NC + lax.axis_index("c")
    base = wid * NB
    pltpu.sync_copy(s_hbm.at[pl.ds(base, NB)], sidx)
    lane16 = lax.iota(jnp.int32, 16)

    xs_bufs = (xs0, xs1)
    xw_bufs = (xw0, xw1)
    wst_bufs = (wst0, wst1)
    wix_bufs = (wix0, wix1)
    sems = (sem0, sem1)
    wsems = (wsem0, wsem1)

    def stage_w(c):
        return pltpu.async_copy(
            w_hbm.at[pl.ds(base + c * C, C), :], wst_bufs[c % 2], wsems[c % 2])

    def fire(c, wcp):
        xs_v, xw_v, sem = xs_bufs[c % 2], xw_bufs[c % 2], sems[c % 2]
        wst, wix = wst_bufs[c % 2], wix_bufs[c % 2]
        wcp.wait()
        for g in range(C // 16):
            rows = g * 16 + lane16
            for j in range(WALK):
                cols = jnp.full((16,), j, jnp.int32)
                wix[j, pl.ds(g * 16, 16)] = plsc.load_gather(wst, [rows, cols])
        cps = [pltpu.async_copy(x_hbm.at[sidx.at[pl.ds(c * C, C)]], xs_v, sem)]
        for j in range(WALK):
            cps.append(pltpu.async_copy(
                x_hbm.at[wix.at[j, pl.ds(0, C)]],
                xw_v.at[pl.ds(j * C, C)], sem))
        return cps

    lane_iota = lax.iota(jnp.int32, 16)

    def compute(c):
        xs_v, xw_v = xs_bufs[c % 2], xw_bufs[c % 2]

        def g_body(g, carry2):
            def l_body(i, accs):
                b = g * 16 + i
                xs = [xs_v[b, pl.ds(k * 16, 16)] for k in range(8)]
                new = []
                for j in range(WALK):
                    r = j * C + b
                    acc = xs[0] * xw_v[r, pl.ds(0, 16)]
                    for k in range(1, 8):
                        acc = acc + xs[k] * xw_v[r, pl.ds(k * 16, 16)]
                    lj = jnp.sum(acc)
                    new.append(jnp.where(lane_iota == i, lj, accs[j]))
                return tuple(new)

            accs = lax.fori_loop(
                0, 16, l_body,
                tuple(jnp.zeros((16,), jnp.float32) for _ in range(WALK)))
            for j in range(WALK):
                lt_v[j, pl.ds(c * C + g * 16, 16)] = accs[j]
            return carry2

        lax.fori_loop(0, C // 16, g_body, 0)

    wcps = [stage_w(0), stage_w(1)]
    inflight = fire(0, wcps[0])
    for c in range(NCHUNK):
        for cp in inflight:
            cp.wait()
        if c + 2 < NCHUNK:
            wcps[c % 2] = stage_w(c + 2)
        if c + 1 < NCHUNK:
            inflight = fire(c + 1, wcps[(c + 1) % 2])
        compute(c)

    stats = []
    for j in range(WALK):
        m = lt_v[j, pl.ds(0, 16)]
        for i in range(1, NB // 16):
            m = jnp.maximum(m, lt_v[j, pl.ds(i * 16, 16)])
        mj = jnp.max(m)
        se = jnp.zeros((16,), jnp.float32)
        for i in range(NB // 16):
            se = se + jnp.exp(lt_v[j, pl.ds(i * 16, 16)] - mj)
        stats.append((mj, jnp.sum(se)))

    def t_body(i, carry3):
        tv = lt_v[0, pl.ds(i * 16, 16)]
        for j in range(1, WALK):
            tv = tv + lt_v[j, pl.ds(i * 16, 16)]
        t_v[pl.ds(i * 16, 16)] = tv
        return carry3

    lax.fori_loop(0, NB // 16, t_body, 0)
    pltpu.sync_copy(t_v, t_hbm.at[0, pl.ds(base, NB)])

    st = jnp.zeros((16,), jnp.float32)
    for j in range(WALK):
        st = jnp.where(lane_iota == j, stats[j][0], st)
        st = jnp.where(lane_iota == (j + 8), stats[j][1], st)
    for q in range(8):
        st_v[0, pl.ds(q * 16, 16)] = st if q == 0 else jnp.zeros(
            (16,), jnp.float32)
    pltpu.sync_copy(st_v, st_hbm.at[wid])


_sc_partial = functools.partial(
    pl.kernel,
    mesh=plsc.VectorSubcoreMesh(core_axis_name="c", subcore_axis_name="s"),
    compiler_params=pltpu.CompilerParams(needs_layout_passes=False),
    out_type=(
        jax.ShapeDtypeStruct((1, B), jnp.float32),
        jax.ShapeDtypeStruct((NW, 1, 128), jnp.float32),
    ),
    scratch_types=[
        pltpu.VMEM((NB,), jnp.int32),
        pltpu.VMEM((C, WALK), jnp.int32),
        pltpu.VMEM((C, WALK), jnp.int32),
        pltpu.VMEM((WALK, C), jnp.int32),
        pltpu.VMEM((WALK, C), jnp.int32),
        pltpu.VMEM((C, D), jnp.float32),
        pltpu.VMEM((C, D), jnp.float32),
        pltpu.VMEM((C * WALK, D), jnp.float32),
        pltpu.VMEM((C * WALK, D), jnp.float32),
        pltpu.VMEM((WALK, NB), jnp.float32),
        pltpu.VMEM((NB,), jnp.float32),
        pltpu.VMEM((1, 128), jnp.float32),
        pltpu.SemaphoreType.DMA,
        pltpu.SemaphoreType.DMA,
        pltpu.SemaphoreType.DMA,
        pltpu.SemaphoreType.DMA,
    ],
)(_sc_body)


def _tc_body(t_ref, st_ref, out_ref):
    st = st_ref[...][:, 0, :]
    m = st[:, 0:WALK]
    se = st[:, 8:8 + WALK]
    gm = jnp.max(m, axis=0, keepdims=True)
    s_all = jnp.sum(se * jnp.exp(m - gm), axis=0, keepdims=True)
    k_const = jnp.sum(gm + jnp.log(s_all))
    out_ref[...] = k_const - t_ref[0]


def kernel(s, w, neg, X):
    del neg
    t, st = _sc_partial(s, w, X)
    return pl.pallas_call(
        _tc_body,
        out_shape=jax.ShapeDtypeStruct((B,), jnp.float32),
    )(t, st)


# speedup vs baseline: 1.1893x; 1.0741x over previous
import functools

import jax
import jax.numpy as jnp
from jax import lax
from jax.experimental import pallas as pl
from jax.experimental.pallas import tpu as pltpu
from jax.experimental.pallas import tpu_sc as plsc

B = 16384
D = 128
WALK = 5
NC = 2
NS = 16
NW = NC * NS
NB = B // NW
C = 64
NCHUNK = NB // C


def _sc_body(s_hbm, w_hbm, x_hbm, t_hbm, st_hbm, sidx, widx,
             xs0, xs1, xw0, xw1, lt_v, t_v, st_v, sem0, sem1):
    wid = lax.axis_index("s") * NC + lax.axis_index("c")
    base = wid * NB
    pltpu.sync_copy(s_hbm.at[pl.ds(base, NB)], sidx)
    pltpu.sync_copy(w_hbm.at[:, pl.ds(base, NB)], widx)
    lane_iota = lax.iota(jnp.int32, 16)

    xs_bufs = (xs0, xs1)
    xw_bufs = (xw0, xw1)
    sems = (sem0, sem1)

    def fire(c):
        xs_v, xw_v, sem = xs_bufs[c % 2], xw_bufs[c % 2], sems[c % 2]
        cps = [pltpu.async_copy(x_hbm.at[sidx.at[pl.ds(c * C, C)]], xs_v, sem)]
        for j in range(WALK):
            cps.append(pltpu.async_copy(
                x_hbm.at[widx.at[j, pl.ds(c * C, C)]],
                xw_v.at[pl.ds(j * C, C)], sem))
        return cps

    def compute(c):
        xs_v, xw_v = xs_bufs[c % 2], xw_bufs[c % 2]

        def g_body(g, carry2):
            def l_body(i, accs):
                b = g * 16 + i
                xs = [xs_v[b, pl.ds(k * 16, 16)] for k in range(8)]
                new = []
                for j in range(WALK):
                    r = j * C + b
                    acc = xs[0] * xw_v[r, pl.ds(0, 16)]
                    for k in range(1, 8):
                        acc = acc + xs[k] * xw_v[r, pl.ds(k * 16, 16)]
                    lj = jnp.sum(acc)
                    new.append(jnp.where(lane_iota == i, lj, accs[j]))
                return tuple(new)

            accs = lax.fori_loop(
                0, 16, l_body,
                tuple(jnp.zeros((16,), jnp.float32) for _ in range(WALK)))
            for j in range(WALK):
                lt_v[j, pl.ds(c * C + g * 16, 16)] = accs[j]
            return carry2

        lax.fori_loop(0, C // 16, g_body, 0)

    inflight = fire(0)
    for c in range(NCHUNK):
        for cp in inflight:
            cp.wait()
        if c + 1 < NCHUNK:
            inflight = fire(c + 1)
        compute(c)

    stats = []
    for j in range(WALK):
        m = lt_v[j, pl.ds(0, 16)]
        for i in range(1, NB // 16):
            m = jnp.maximum(m, lt_v[j, pl.ds(i * 16, 16)])
        mj = jnp.max(m)
        se = jnp.zeros((16,), jnp.float32)
        for i in range(NB // 16):
            se = se + jnp.exp(lt_v[j, pl.ds(i * 16, 16)] - mj)
        stats.append((mj, jnp.sum(se)))

    def t_body(i, carry3):
        tv = lt_v[0, pl.ds(i * 16, 16)]
        for j in range(1, WALK):
            tv = tv + lt_v[j, pl.ds(i * 16, 16)]
        t_v[pl.ds(i * 16, 16)] = tv
        return carry3

    lax.fori_loop(0, NB // 16, t_body, 0)
    pltpu.sync_copy(t_v, t_hbm.at[0, pl.ds(base, NB)])

    st = jnp.zeros((16,), jnp.float32)
    for j in range(WALK):
        st = jnp.where(lane_iota == j, stats[j][0], st)
        st = jnp.where(lane_iota == (j + 8), stats[j][1], st)
    for q in range(8):
        st_v[0, pl.ds(q * 16, 16)] = st if q == 0 else jnp.zeros(
            (16,), jnp.float32)
    pltpu.sync_copy(st_v, st_hbm.at[wid])


_sc_partial = functools.partial(
    pl.kernel,
    mesh=plsc.VectorSubcoreMesh(core_axis_name="c", subcore_axis_name="s"),
    compiler_params=pltpu.CompilerParams(needs_layout_passes=False),
    out_type=(
        jax.ShapeDtypeStruct((1, B), jnp.float32),
        jax.ShapeDtypeStruct((NW, 1, 128), jnp.float32),
    ),
    scratch_types=[
        pltpu.VMEM((NB,), jnp.int32),
        pltpu.VMEM((WALK, NB), jnp.int32),
        pltpu.VMEM((C, D), jnp.float32),
        pltpu.VMEM((C, D), jnp.float32),
        pltpu.VMEM((C * WALK, D), jnp.float32),
        pltpu.VMEM((C * WALK, D), jnp.float32),
        pltpu.VMEM((WALK, NB), jnp.float32),
        pltpu.VMEM((NB,), jnp.float32),
        pltpu.VMEM((1, 128), jnp.float32),
        pltpu.SemaphoreType.DMA,
        pltpu.SemaphoreType.DMA,
    ],
)(_sc_body)


def _tc_body(t_ref, st_ref, out_ref):
    st = st_ref[...][:, 0, :]
    m = st[:, 0:WALK]
    se = st[:, 8:8 + WALK]
    gm = jnp.max(m, axis=0, keepdims=True)
    s_all = jnp.sum(se * jnp.exp(m - gm), axis=0, keepdims=True)
    k_const = jnp.sum(gm + jnp.log(s_all))
    out_ref[...] = k_const - t_ref[0]


def kernel(s, w, neg, X):
    del neg
    t, st = _sc_partial(s, w.T, X)
    return pl.pallas_call(
        _tc_body,
        out_shape=jax.ShapeDtypeStruct((B,), jnp.float32),
    )(t, st)


# speedup vs baseline: 1.2523x; 1.0530x over previous
import functools

import jax
import jax.numpy as jnp
from jax import lax
from jax.experimental import pallas as pl
from jax.experimental.pallas import tpu as pltpu
from jax.experimental.pallas import tpu_sc as plsc

B = 16384
D = 128
WALK = 5
NC = 2
NS = 16
NW = NC * NS
NB = B // NW
C = 64
NCHUNK = NB // C


def _sc_body(s_hbm, w_hbm, x_hbm, t_hbm, st_hbm, sidx, widx,
             xs0, xs1, xw0, xw1, lt_v, t_v, st_v, sem0, sem1):
    wid = lax.axis_index("s") * NC + lax.axis_index("c")
    base = wid * NB
    pltpu.sync_copy(s_hbm.at[pl.ds(base, NB)], sidx)
    pltpu.sync_copy(w_hbm.at[:, pl.ds(base, NB)], widx)
    lane_iota = lax.iota(jnp.int32, 16)

    xs_bufs = (xs0, xs1)
    xw_bufs = (xw0, xw1)
    sems = (sem0, sem1)

    def fire(c, p):
        xs_v, xw_v, sem = xs_bufs[p], xw_bufs[p], sems[p]
        pltpu.async_copy(x_hbm.at[sidx.at[pl.ds(c * C, C)]], xs_v, sem)
        for j in range(WALK):
            pltpu.async_copy(
                x_hbm.at[widx.at[j, pl.ds(c * C, C)]],
                xw_v.at[pl.ds(j * C, C)], sem)

    def wait(c, p):
        xs_v, xw_v, sem = xs_bufs[p], xw_bufs[p], sems[p]
        pltpu.make_async_copy(
            x_hbm.at[sidx.at[pl.ds(c * C, C)]], xs_v, sem).wait()
        for j in range(WALK):
            pltpu.make_async_copy(
                x_hbm.at[widx.at[j, pl.ds(c * C, C)]],
                xw_v.at[pl.ds(j * C, C)], sem).wait()

    def compute(c, p):
        xs_v, xw_v = xs_bufs[p], xw_bufs[p]

        def g_body(g, carry2):
            def l_body(i, accs):
                b = g * 16 + i
                xs = [xs_v[b, pl.ds(k * 16, 16)] for k in range(8)]
                new = []
                for j in range(WALK):
                    r = j * C + b
                    acc = xs[0] * xw_v[r, pl.ds(0, 16)]
                    for k in range(1, 8):
                        acc = acc + xs[k] * xw_v[r, pl.ds(k * 16, 16)]
                    lj = jnp.sum(acc)
                    new.append(jnp.where(lane_iota == i, lj, accs[j]))
                return tuple(new)

            accs = lax.fori_loop(
                0, 16, l_body,
                tuple(jnp.zeros((16,), jnp.float32) for _ in range(WALK)))
            for j in range(WALK):
                lt_v[j, pl.ds(c * C + g * 16, 16)] = accs[j]
            return carry2

        lax.fori_loop(0, C // 16, g_body, 0)

    fire(0, 0)

    def chunk_body(c, carry4):
        def step(p):
            def go(_):
                wait(c, p)

                @pl.when(c + 1 < NCHUNK)
                def _fire_next():
                    fire(c + 1, 1 - p)

                compute(c, p)
                return 0
            return go

        return lax.cond(c % 2 == 0, step(0), step(1), carry4)

    lax.fori_loop(0, NCHUNK, chunk_body, 0)

    stats = []
    for j in range(WALK):
        m = lt_v[j, pl.ds(0, 16)]
        for i in range(1, NB // 16):
            m = jnp.maximum(m, lt_v[j, pl.ds(i * 16, 16)])
        mj = jnp.max(m)
        se = jnp.zeros((16,), jnp.float32)
        for i in range(NB // 16):
            se = se + jnp.exp(lt_v[j, pl.ds(i * 16, 16)] - mj)
        stats.append((mj, jnp.sum(se)))

    def t_body(i, carry3):
        tv = lt_v[0, pl.ds(i * 16, 16)]
        for j in range(1, WALK):
            tv = tv + lt_v[j, pl.ds(i * 16, 16)]
        t_v[pl.ds(i * 16, 16)] = tv
        return carry3

    lax.fori_loop(0, NB // 16, t_body, 0)
    pltpu.sync_copy(t_v, t_hbm.at[0, pl.ds(base, NB)])

    st = jnp.zeros((16,), jnp.float32)
    for j in range(WALK):
        st = jnp.where(lane_iota == j, stats[j][0], st)
        st = jnp.where(lane_iota == (j + 8), stats[j][1], st)
    for q in range(8):
        st_v[0, pl.ds(q * 16, 16)] = st if q == 0 else jnp.zeros(
            (16,), jnp.float32)
    pltpu.sync_copy(st_v, st_hbm.at[wid])


_sc_partial = functools.partial(
    pl.kernel,
    mesh=plsc.VectorSubcoreMesh(core_axis_name="c", subcore_axis_name="s"),
    compiler_params=pltpu.CompilerParams(needs_layout_passes=False),
    out_type=(
        jax.ShapeDtypeStruct((1, B), jnp.float32),
        jax.ShapeDtypeStruct((NW, 1, 128), jnp.float32),
    ),
    scratch_types=[
        pltpu.VMEM((NB,), jnp.int32),
        pltpu.VMEM((WALK, NB), jnp.int32),
        pltpu.VMEM((C, D), jnp.float32),
        pltpu.VMEM((C, D), jnp.float32),
        pltpu.VMEM((C * WALK, D), jnp.float32),
        pltpu.VMEM((C * WALK, D), jnp.float32),
        pltpu.VMEM((WALK, NB), jnp.float32),
        pltpu.VMEM((NB,), jnp.float32),
        pltpu.VMEM((1, 128), jnp.float32),
        pltpu.SemaphoreType.DMA,
        pltpu.SemaphoreType.DMA,
    ],
)(_sc_body)


def _tc_body(t_ref, st_ref, out_ref):
    st = st_ref[...][:, 0, :]
    m = st[:, 0:WALK]
    se = st[:, 8:8 + WALK]
    gm = jnp.max(m, axis=0, keepdims=True)
    s_all = jnp.sum(se * jnp.exp(m - gm), axis=0, keepdims=True)
    k_const = jnp.sum(gm + jnp.log(s_all))
    out_ref[...] = k_const - t_ref[0]


def kernel(s, w, neg, X):
    del neg
    t, st = _sc_partial(s, w.T, X)
    return pl.pallas_call(
        _tc_body,
        out_shape=jax.ShapeDtypeStruct((B,), jnp.float32),
    )(t, st)
